# Initial kernel scaffold; baseline (speedup 1.0000x reference)
import jax
import jax.numpy as jnp
from jax.experimental import pallas as pl


def kernel(nodes0, self_idx1, neigh1, self_idx2, neigh2, table, W1, b1, W2, b2):
    raise NotImplementedError("write your pallas kernel here")


# speedup vs baseline: 3.7061x; 3.7061x over previous
import functools

import jax
import jax.numpy as jnp
from jax import lax
from jax.experimental import pallas as pl
from jax.experimental.pallas import tpu as pltpu
from jax.experimental.pallas import tpu_sc as plsc

N_NODES, D = 50000, 256
N0, N1, N2, S = 262144, 16384, 1024, 16
NC, NS = 2, 16
NW = NC * NS
LANES = 16
NB = 8
CHUNK = NB * S


def _make_agg(n_out, n_tab, compose):
    rows_w = n_out // NW
    iters = rows_w // NB
    mesh = plsc.VectorSubcoreMesh(core_axis_name="c", subcore_axis_name="s")
    scale = 1.0 / S

    def body(nodes0_hbm, sidx_hbm, nidx_hbm, tab_hbm, selfv_hbm, neigh_hbm,
             nidx_v, sidx_v, sg_v, g_v, rows_v, srows_v, acc_v,
             sem_i, sem_g, sem_s):
        wid = lax.axis_index("s") * NC + lax.axis_index("c")
        base = wid * rows_w
        pltpu.sync_copy(nidx_hbm.at[pl.ds(base * S, rows_w * S)], nidx_v)
        pltpu.sync_copy(sidx_hbm.at[pl.ds(base, rows_w)], sidx_v)
        if compose:
            def comp_self(j, _):
                pltpu.async_copy(
                    nodes0_hbm.at[sidx_v.at[pl.ds(j * LANES * 8, LANES * 8)]],
                    sg_v.at[pl.ds(j * LANES * 8, LANES * 8)], sem_i).wait()
                return 0
            lax.fori_loop(0, rows_w // (LANES * 8), comp_self, 0)
            sidx_src = sg_v
        else:
            sidx_src = sidx_v

        def it_body(i, _):
            if compose:
                pltpu.async_copy(
                    nodes0_hbm.at[nidx_v.at[pl.ds(i * CHUNK, CHUNK)]],
                    g_v, sem_i).wait()
                gsrc = g_v
            else:
                gsrc = nidx_v.at[pl.ds(i * CHUNK, CHUNK)]
            cp_rows = pltpu.async_copy(tab_hbm.at[gsrc], rows_v, sem_g)
            cp_self = pltpu.async_copy(
                tab_hbm.at[sidx_src.at[pl.ds(i * NB, NB)]], srows_v, sem_s)
            cp_rows.wait()
            def grp(gi, _):
                def col(ci, _):
                    cs = ci * LANES
                    acc = rows_v[gi * S, pl.ds(cs, LANES)]
                    for r in range(1, S):
                        acc = acc + rows_v[gi * S + r, pl.ds(cs, LANES)]
                    acc_v[gi, pl.ds(cs, LANES)] = acc * scale
                    return 0
                lax.fori_loop(0, D // LANES, col, 0)
                return 0
            lax.fori_loop(0, NB, grp, 0)
            cp_self.wait()
            pltpu.sync_copy(acc_v, neigh_hbm.at[pl.ds(base + i * NB, NB), :])
            pltpu.sync_copy(srows_v, selfv_hbm.at[pl.ds(base + i * NB, NB), :])
            return 0

        lax.fori_loop(0, iters, it_body, 0)

    f32 = jnp.float32
    return pl.kernel(
        body,
        out_type=(jax.ShapeDtypeStruct((n_out, D), f32),
                  jax.ShapeDtypeStruct((n_out, D), f32)),
        mesh=mesh,
        scratch_types=[
            pltpu.VMEM((rows_w * S,), jnp.int32),
            pltpu.VMEM((rows_w,), jnp.int32),
            pltpu.VMEM((rows_w,), jnp.int32),
            pltpu.VMEM((CHUNK,), jnp.int32),
            pltpu.VMEM((CHUNK, D), f32),
            pltpu.VMEM((NB, D), f32),
            pltpu.VMEM((NB, D), f32),
            pltpu.SemaphoreType.DMA,
            pltpu.SemaphoreType.DMA,
            pltpu.SemaphoreType.DMA,
        ],
    )


def _mm_body(sv_ref, nb_ref, wa_ref, wb_ref, b_ref, o_ref):
    acc = jnp.dot(sv_ref[...], wa_ref[...], preferred_element_type=jnp.float32)
    acc = acc + jnp.dot(nb_ref[...], wb_ref[...], preferred_element_type=jnp.float32)
    o_ref[...] = jnp.maximum(acc + b_ref[...], 0.0)


def _encoder(selfv, neigh, W, b, bm):
    n = selfv.shape[0]
    return pl.pallas_call(
        _mm_body,
        grid=(n // bm,),
        in_specs=[
            pl.BlockSpec((bm, D), lambda i: (i, 0)),
            pl.BlockSpec((bm, D), lambda i: (i, 0)),
            pl.BlockSpec((D, D), lambda i: (0, 0)),
            pl.BlockSpec((D, D), lambda i: (0, 0)),
            pl.BlockSpec((1, D), lambda i: (0, 0)),
        ],
        out_specs=pl.BlockSpec((bm, D), lambda i: (i, 0)),
        out_shape=jax.ShapeDtypeStruct((n, D), jnp.float32),
    )(selfv, neigh, W[:D], W[D:], b.reshape(1, D))


_agg1 = _make_agg(N1, N_NODES, compose=True)
_agg2 = _make_agg(N2, N1, compose=False)


@jax.jit
def kernel(nodes0, self_idx1, neigh1, self_idx2, neigh2, table, W1, b1, W2, b2):
    nodes0 = nodes0.astype(jnp.int32)
    selfv1, neigh1m = _agg1(nodes0, self_idx1.astype(jnp.int32),
                            neigh1.astype(jnp.int32).reshape(-1), table)
    h1 = _encoder(selfv1, neigh1m, W1, b1, 2048)
    selfv2, neigh2m = _agg2(nodes0, self_idx2.astype(jnp.int32),
                            neigh2.astype(jnp.int32).reshape(-1), h1)
    h2 = _encoder(selfv2, neigh2m, W2, b2, 1024)
    return h2


# speedup vs baseline: 8.3174x; 2.2442x over previous
import jax
import jax.numpy as jnp
from jax import lax
from jax.experimental import pallas as pl
from jax.experimental.pallas import tpu as pltpu
from jax.experimental.pallas import tpu_sc as plsc

N_NODES, D = 50000, 256
N0, N1, N2, S = 262144, 16384, 1024, 16
NC, NS = 2, 16
NW = NC * NS
LANES = 16
NB = 8
CHUNK = NB * S
WAVE = 16


def _make_agg(n_out, compose):
    rows_w = n_out // NW
    iters = rows_w // NB
    mesh = plsc.VectorSubcoreMesh(core_axis_name="c", subcore_axis_name="s")
    scale = 1.0 / S

    def body(nodes0_hbm, sidx_hbm, nidx_hbm, tab_hbm, selfv_hbm, neigh_hbm,
             nidx_v, sidx_v, sg_v, g_all, rows_v, srows_v, acc_v,
             sem_i, sem_g, sem_s, sem_oa, sem_os):
        wid = lax.axis_index("s") * NC + lax.axis_index("c")
        base = wid * rows_w
        pltpu.sync_copy(nidx_hbm.at[pl.ds(base * S, rows_w * S)], nidx_v)
        pltpu.sync_copy(sidx_hbm.at[pl.ds(base, rows_w)], sidx_v)

        if compose:
            n_chunks = rows_w * S // CHUNK

            def comp_n(j):
                return pltpu.async_copy(
                    nodes0_hbm.at[nidx_v.at[pl.ds(j * CHUNK, CHUNK)]],
                    g_all.at[pl.ds(j * CHUNK, CHUNK)], sem_i)

            def wave_body(w, _):
                def fire(j, _):
                    comp_n(w * WAVE + j)
                    return 0
                lax.fori_loop(0, WAVE, fire, 0)
                def drain(j, _):
                    pltpu.make_async_copy(
                        nodes0_hbm.at[nidx_v.at[pl.ds(j * CHUNK, CHUNK)]],
                        g_all.at[pl.ds(j * CHUNK, CHUNK)], sem_i).wait()
                    return 0
                lax.fori_loop(0, WAVE, drain, 0)
                return 0
            lax.fori_loop(0, max(n_chunks // WAVE, 1), wave_body, 0)
            for j in range(0, rows_w, CHUNK):
                n = min(CHUNK, rows_w - j)
                pltpu.async_copy(nodes0_hbm.at[sidx_v.at[pl.ds(j, n)]],
                                 sg_v.at[pl.ds(j, n)], sem_i).wait()
            g_src, s_src = g_all, sg_v
        else:
            g_src, s_src = nidx_v, sidx_v

        def rows_cp(i, b):
            return pltpu.make_async_copy(
                tab_hbm.at[g_src.at[pl.ds(i * CHUNK, CHUNK)]],
                rows_v.at[b], sem_g)

        def self_cp(i, b4):
            return pltpu.make_async_copy(
                tab_hbm.at[s_src.at[pl.ds(i * NB, NB)]],
                srows_v.at[b4], sem_s)

        def out_acc_cp(i, b4):
            return pltpu.make_async_copy(
                acc_v.at[b4], neigh_hbm.at[pl.ds(base + i * NB, NB), :], sem_oa)

        def out_self_cp(i, b4):
            return pltpu.make_async_copy(
                srows_v.at[b4], selfv_hbm.at[pl.ds(base + i * NB, NB), :], sem_os)

        def accumulate(b2, b4):
            def grp(gi, _):
                def col(ci, _):
                    cs = ci * 2 * LANES
                    a0 = rows_v[b2, gi * S, pl.ds(cs, LANES)]
                    a1 = rows_v[b2, gi * S, pl.ds(cs + LANES, LANES)]
                    for r in range(1, S):
                        a0 = a0 + rows_v[b2, gi * S + r, pl.ds(cs, LANES)]
                        a1 = a1 + rows_v[b2, gi * S + r, pl.ds(cs + LANES, LANES)]
                    acc_v[b4, gi, pl.ds(cs, LANES)] = a0 * scale
                    acc_v[b4, gi, pl.ds(cs + LANES, LANES)] = a1 * scale
                    return 0
                lax.fori_loop(0, D // (2 * LANES), col, 0)
                return 0
            lax.fori_loop(0, NB, grp, 0)

        rows_cp(0, 0).start()
        self_cp(0, 0).start()
        rows_cp(1, 1).start()
        self_cp(1, 1).start()

        def outer(o, _):
            i0 = o * 4
            for u in range(4):
                i = i0 + u
                b2 = u % 2
                @pl.when(i >= 2)
                def _():
                    out_acc_cp(i - 2, (u + 2) % 4).wait()
                    out_self_cp(i - 2, (u + 2) % 4).wait()
                rows_cp(i, b2).wait()
                self_cp(i, u).wait()
                accumulate(b2, u)
                out_acc_cp(i, u).start()
                out_self_cp(i, u).start()
                @pl.when(i + 2 < iters)
                def _():
                    rows_cp(i + 2, b2).start()
                    self_cp(i + 2, (u + 2) % 4).start()
            return 0

        lax.fori_loop(0, iters // 4, outer, 0)
        out_acc_cp(iters - 2, (iters - 2) % 4).wait()
        out_self_cp(iters - 2, (iters - 2) % 4).wait()
        out_acc_cp(iters - 1, (iters - 1) % 4).wait()
        out_self_cp(iters - 1, (iters - 1) % 4).wait()

    f32 = jnp.float32
    return pl.kernel(
        body,
        out_type=(jax.ShapeDtypeStruct((n_out, D), f32),
                  jax.ShapeDtypeStruct((n_out, D), f32)),
        mesh=mesh,
        scratch_types=[
            pltpu.VMEM((rows_w * S,), jnp.int32),
            pltpu.VMEM((rows_w,), jnp.int32),
            pltpu.VMEM((rows_w,), jnp.int32),
            pltpu.VMEM((rows_w * S,), jnp.int32),
            pltpu.VMEM((2, CHUNK, D), f32),
            pltpu.VMEM((4, NB, D), f32),
            pltpu.VMEM((4, NB, D), f32),
            pltpu.SemaphoreType.DMA,
            pltpu.SemaphoreType.DMA,
            pltpu.SemaphoreType.DMA,
            pltpu.SemaphoreType.DMA,
            pltpu.SemaphoreType.DMA,
        ],
    )


def _mm_body(sv_ref, nb_ref, wa_ref, wb_ref, b_ref, o_ref):
    acc = jnp.dot(sv_ref[...], wa_ref[...], preferred_element_type=jnp.float32)
    acc = acc + jnp.dot(nb_ref[...], wb_ref[...], preferred_element_type=jnp.float32)
    o_ref[...] = jnp.maximum(acc + b_ref[...], 0.0)


def _encoder(selfv, neigh, W, b, bm):
    n = selfv.shape[0]
    return pl.pallas_call(
        _mm_body,
        grid=(n // bm,),
        in_specs=[
            pl.BlockSpec((bm, D), lambda i: (i, 0)),
            pl.BlockSpec((bm, D), lambda i: (i, 0)),
            pl.BlockSpec((D, D), lambda i: (0, 0)),
            pl.BlockSpec((D, D), lambda i: (0, 0)),
            pl.BlockSpec((1, D), lambda i: (0, 0)),
        ],
        out_specs=pl.BlockSpec((bm, D), lambda i: (i, 0)),
        out_shape=jax.ShapeDtypeStruct((n, D), jnp.float32),
    )(selfv, neigh, W[:D], W[D:], b.reshape(1, D))


_agg1 = _make_agg(N1, compose=True)
_agg2 = _make_agg(N2, compose=False)


@jax.jit
def kernel(nodes0, self_idx1, neigh1, self_idx2, neigh2, table, W1, b1, W2, b2):
    nodes0 = nodes0.astype(jnp.int32)
    selfv1, neigh1m = _agg1(nodes0, self_idx1.astype(jnp.int32),
                            neigh1.astype(jnp.int32).reshape(-1), table)
    h1 = _encoder(selfv1, neigh1m, W1, b1, 2048)
    selfv2, neigh2m = _agg2(nodes0, self_idx2.astype(jnp.int32),
                            neigh2.astype(jnp.int32).reshape(-1), h1)
    h2 = _encoder(selfv2, neigh2m, W2, b2, 1024)
    return h2
